# baseline (speedup 1.0000x reference)
import functools

import jax
import jax.numpy as jnp
from jax import lax
from jax.experimental import pallas as pl
from jax.experimental.pallas import tpu as pltpu
from jax.experimental.pallas import tpu_sc as plsc

ROWS = 32
V = 1_000_000
WC = 4096
NWIN = V // WC
REMC = V - NWIN * WC
CH = 512
CPW = WC // CH
NCHUNK = NWIN * CPW + 1
NV_CM = (NCHUNK + 15) // 16
CMPAD = NV_CM * 16
CAP = 1024
K = 50
NEG = -3.0e38
BIG = 2**30
TEMP = 0.7
TOPP = 0.9
L = 16

SW = 2048
SCH = 248
STRIPE = SCH * CH
SNW = STRIPE // SW
S7NW = 54
S7W = 232


def _sampler_body(logits_hbm, rr_hbm, out_hbm,
                  blk, blk2, cmax, cmx2, cmloc, shared, cbuf, cval, cidx,
                  sval, sidx, pbuf, rrv, outv):
    sid = lax.axis_index("s")
    wid = lax.axis_index("c") * 16 + sid
    rb = (wid // 8) * 8
    q = wid % 8
    iota = lax.iota(jnp.int32, L)
    negv = jnp.full((L,), NEG, jnp.float32)
    bigv = jnp.full((L,), BIG, jnp.int32)

    pltpu.sync_copy(rr_hbm, rrv)

    j = sid % 8
    lrb = (sid // 8) * 8
    sbase = j * STRIPE
    nwin_j = jnp.where(j < 7, SNW, S7NW)

    def initcm(i, _):
        cmloc[pl.ds(i * L, L)] = negv
        return 0

    lax.fori_loop(0, 2048 // L, initcm, 0)

    def compute_win2(buf, g):
        for i in range(SW // CH):
            cb = i * CH
            gci = g * (SW // CH) + i
            for r in range(8):
                acc = buf[r, pl.ds(cb, L)]
                for v in range(1, CH // L):
                    acc = jnp.maximum(acc, buf[r, pl.ds(cb + v * L, L)])
                cm = jnp.max(acc)
                plsc.store_scatter(
                    cmloc,
                    [jnp.zeros((L,), jnp.int32) + (r * 256 + gci)],
                    jnp.zeros((L,), jnp.float32) + cm, mask=iota == 0)

    def window_body(g, _):
        pltpu.sync_copy(
            logits_hbm.at[pl.ds(rb, 8), pl.ds(sbase + g * SW, SW)], blk)
        compute_win2(blk, g)
        return 0

    lax.fori_loop(0, nwin_j, window_body, 0)

    @pl.when(j == 7)
    def _():
        pltpu.sync_copy(
            logits_hbm.at[pl.ds(rb, 8), pl.ds(NWIN * WC, REMC)], blk2)
        for r in range(8):
            acc = blk2[r, pl.ds(0, L)]
            for v in range(1, REMC // L):
                acc = jnp.maximum(acc, blk2[r, pl.ds(v * L, L)])
            cm = jnp.max(acc)
            plsc.store_scatter(
                cmloc,
                [jnp.zeros((L,), jnp.int32) + (r * 256 + S7NW * (SW // CH))],
                jnp.zeros((L,), jnp.float32) + cm, mask=iota == 0)

    for r in range(8):
        def c_lt7(_, r=r):
            pltpu.sync_copy(
                cmloc.at[pl.ds(r * 256, SCH)],
                shared.at[pl.ds((lrb + r) * 2048 + j * SCH, SCH)])
            return 0

        def c_eq7(_, r=r):
            pltpu.sync_copy(
                cmloc.at[pl.ds(r * 256, S7W)],
                shared.at[pl.ds((lrb + r) * 2048 + 7 * SCH, S7W)])
            return 0

        lax.cond(j < 7, c_lt7, c_eq7, 0)

    plsc.subcore_barrier()

    pltpu.sync_copy(shared.at[pl.ds(sid * 2048, 2048)], cmax)

    def copy_body(i, _):
        cmx2[pl.ds(i * L, L)] = cmax[pl.ds(i * L, L)]
        return 0

    lax.fori_loop(0, NV_CM, copy_body, 0)

    def tau_iter(t, _):
        def sweep(i, a):
            return jnp.maximum(a, cmx2[pl.ds(i * L, L)])

        a = lax.fori_loop(0, NV_CM, sweep, negv)
        vstar = jnp.max(a)

        def mask_out(i, _):
            vv = cmx2[pl.ds(i * L, L)]
            cmx2[pl.ds(i * L, L)] = jnp.where(vv == vstar, negv, vv)
            return 0

        lax.fori_loop(0, NV_CM, mask_out, 0)
        return vstar

    tau = lax.fori_loop(0, K, tau_iter, jnp.float32(NEG))

    def init_cand(i, _):
        cval[pl.ds(i * L, L)] = negv
        cidx[pl.ds(i * L, L)] = bigv
        return 0

    lax.fori_loop(0, CAP // L, init_cand, 0)

    def append(ref, nv_, col0, cnt):
        def vreg(j, cnt):
            vv = ref[q, pl.ds(j * L, L)]
            m = vv >= tau
            mi = m.astype(jnp.int32)
            pos = cnt + plsc.cumsum(mi) - 1
            okm = m & (pos < CAP)
            gidx = col0 + j * L + iota
            plsc.store_scatter(cval, [pos], vv / TEMP, mask=okm)
            plsc.store_scatter(cidx, [pos], gidx, mask=okm)
            return cnt + jnp.sum(mi)

        return lax.fori_loop(0, nv_, vreg, cnt)

    def chunkb_vreg(i, cnt):
        cmv = cmax[pl.ds(i * L, L)]
        anyp = jnp.max(cmv)

        def scan_lanes(cnt):
            for lane in range(L):
                cml = cmv[lane]
                c = i * L + lane

                def do_full(cnt, c=c):
                    pltpu.sync_copy(
                        logits_hbm.at[pl.ds(rb, 8), pl.ds(c * CH, CH)], cbuf)
                    return append(cbuf, CH // L, c * CH, cnt)

                def do_rem(cnt):
                    pltpu.sync_copy(
                        logits_hbm.at[pl.ds(rb, 8), pl.ds(NWIN * WC, REMC)],
                        blk2)
                    return append(blk2, REMC // L, NWIN * WC, cnt)

                def fetch(cnt, c=c, do_full=do_full, do_rem=do_rem):
                    return lax.cond(c < NCHUNK - 1, do_full, do_rem, cnt)

                cnt = lax.cond(cml >= tau, fetch, lambda cnt: cnt, cnt)
            return cnt

        return lax.cond(anyp >= tau, scan_lanes, lambda cnt: cnt, cnt)

    cnt = lax.fori_loop(0, NV_CM, chunkb_vreg, jnp.int32(0))

    nv = (jnp.minimum(cnt, CAP) + (L - 1)) // L

    def ext(t, _):
        def sweep(i, a):
            return jnp.maximum(a, cval[pl.ds(i * L, L)])

        a = lax.fori_loop(0, nv, sweep, negv)
        vstar = jnp.max(a)

        def sweep2(i, a):
            vv = cval[pl.ds(i * L, L)]
            ix = cidx[pl.ds(i * L, L)]
            return jnp.minimum(a, jnp.where(vv == vstar, ix, bigv))

        iacc = lax.fori_loop(0, nv, sweep2, bigv)
        istar = jnp.min(iacc)

        def sweep3(i, _):
            vv = cval[pl.ds(i * L, L)]
            ix = cidx[pl.ds(i * L, L)]
            kill = (vv == vstar) & (ix == istar)
            cval[pl.ds(i * L, L)] = jnp.where(kill, negv, vv)
            return 0

        lax.fori_loop(0, nv, sweep3, 0)
        tv = jnp.zeros((L,), jnp.int32) + t
        plsc.store_scatter(sval, [tv],
                           jnp.zeros((L,), jnp.float32) + vstar,
                           mask=iota == 0)
        plsc.store_scatter(sidx, [tv],
                           jnp.zeros((L,), jnp.int32) + istar,
                           mask=iota == 0)
        return 0

    sval[pl.ds(48, L)] = negv
    sidx[pl.ds(48, L)] = bigv
    lax.fori_loop(0, K, ext, 0)

    m1 = sval[pl.ds(0, L)][0]
    evs = []
    s1acc = jnp.zeros((L,), jnp.float32)
    for b in range(4):
        e = jnp.exp(sval[pl.ds(b * L, L)] - m1)
        evs.append(e)
        s1acc = s1acc + e
    S1 = jnp.sum(s1acc)

    carry = jnp.float32(0.0)
    nkeep = jnp.int32(1)
    for b in range(4):
        cs = plsc.cumsum(evs[b] / S1) + carry
        carry = jnp.max(cs)
        tnum = b * L + iota
        nkeep = nkeep + jnp.sum(((cs <= TOPP) & (tnum < K - 1)).astype(jnp.int32))

    s2acc = jnp.zeros((L,), jnp.float32)
    eks = []
    for b in range(4):
        keep = (b * L + iota) < nkeep
        ek = jnp.where(keep, evs[b], jnp.float32(0.0))
        eks.append(ek)
        s2acc = s2acc + ek
    S2 = jnp.sum(s2acc)
    for b in range(4):
        pbuf[pl.ds(b * L, L)] = eks[b] / S2

    rrvv = rrv[pl.ds((wid // L) * L, L)]
    rr = jnp.max(jnp.where(iota == wid % L, rrvv, jnp.float32(NEG)))

    def ansb(t, ans):
        itv = sidx[pl.ds((t // L) * L, L)]
        it = jnp.min(jnp.where(iota == t % L, itv, bigv))
        acc = jnp.zeros((L,), jnp.float32)
        for b in range(4):
            pv = pbuf[pl.ds(b * L, L)]
            iv = sidx[pl.ds(b * L, L)]
            acc = acc + jnp.where(iv <= it, pv, jnp.float32(0.0))
        P = jnp.sum(acc)
        return jnp.where(P > rr, jnp.minimum(ans, it), ans)

    ans = lax.fori_loop(0, K, ansb, jnp.int32(V))

    outv[...] = jnp.zeros((L,), jnp.int32) + ans
    pltpu.sync_copy(outv, out_hbm.at[wid])


@jax.jit
def _sampler_sc(logits, rr_flat):
    f = functools.partial(
        pl.kernel,
        out_type=jax.ShapeDtypeStruct((ROWS, L), jnp.int32),
        mesh=plsc.VectorSubcoreMesh(core_axis_name="c", subcore_axis_name="s"),
        compiler_params=pltpu.CompilerParams(needs_layout_passes=False,
                                             use_tc_tiling_on_sc=True),
        scratch_types=[
            pltpu.VMEM((8, SW), jnp.float32),
            pltpu.VMEM((8, REMC), jnp.float32),
            pltpu.VMEM((2048,), jnp.float32),
            pltpu.VMEM((2048,), jnp.float32),
            pltpu.VMEM((2048,), jnp.float32),
            pltpu.VMEM_SHARED((32768,), jnp.float32),
            pltpu.VMEM((8, CH), jnp.float32),
            pltpu.VMEM((CAP,), jnp.float32),
            pltpu.VMEM((CAP,), jnp.int32),
            pltpu.VMEM((64,), jnp.float32),
            pltpu.VMEM((64,), jnp.int32),
            pltpu.VMEM((64,), jnp.float32),
            pltpu.VMEM((ROWS,), jnp.float32),
            pltpu.VMEM((L,), jnp.int32),
        ],
    )(_sampler_body)
    return f(logits, rr_flat)


def kernel(logits, rr):
    out16 = _sampler_sc(logits, rr.reshape(-1))
    return out16[:, :1]


# speedup vs baseline: 1.2058x; 1.2058x over previous
import functools

import jax
import jax.numpy as jnp
from jax import lax
from jax.experimental import pallas as pl
from jax.experimental.pallas import tpu as pltpu
from jax.experimental.pallas import tpu_sc as plsc

ROWS = 32
V = 1_000_000
WC = 4096
NWIN = V // WC
REMC = V - NWIN * WC
CH = 512
CPW = WC // CH
NCHUNK = NWIN * CPW + 1
NV_CM = (NCHUNK + 15) // 16
CMPAD = NV_CM * 16
CAP = 1024
K = 50
NEG = -3.0e38
BIG = 2**30
TEMP = 0.7
TOPP = 0.9
L = 16

SW = 2048
SCH = 248
STRIPE = SCH * CH
SNW = STRIPE // SW
S7NW = 54
S7W = 232


def _sampler_body(logits_hbm, rr_hbm, out_hbm,
                  blk, blkB, blk2, cmax, cmx2, cmloc, shared, cbuf, cval,
                  cidx, sval, sidx, pbuf, rrv, outv, semA, semB):
    sid = lax.axis_index("s")
    wid = lax.axis_index("c") * 16 + sid
    rb = (wid // 8) * 8
    q = wid % 8
    iota = lax.iota(jnp.int32, L)
    negv = jnp.full((L,), NEG, jnp.float32)
    bigv = jnp.full((L,), BIG, jnp.int32)

    pltpu.sync_copy(rr_hbm, rrv)

    j = sid % 8
    lrb = (sid // 8) * 8
    sbase = j * STRIPE
    nwin_j = jnp.where(j < 7, SNW, S7NW)

    def initcm(i, _):
        cmloc[pl.ds(i * L, L)] = negv
        return 0

    lax.fori_loop(0, 2048 // L, initcm, 0)

    def compute_win2(buf, g):
        def chunk_i(i, _):
            cb = i * CH
            gci = g * (SW // CH) + i
            for r in range(8):
                acc = buf[r, pl.ds(cb, L)]
                for v in range(1, CH // L):
                    acc = jnp.maximum(acc, buf[r, pl.ds(cb + v * L, L)])
                cm = jnp.max(acc)
                plsc.store_scatter(
                    cmloc,
                    [jnp.zeros((L,), jnp.int32) + (r * 256 + gci)],
                    jnp.zeros((L,), jnp.float32) + cm, mask=iota == 0)
            return 0

        lax.fori_loop(0, SW // CH, chunk_i, 0)

    def win_src(g):
        return logits_hbm.at[pl.ds(rb, 8), pl.ds(sbase + g * SW, SW)]

    pltpu.async_copy(win_src(0), blk, semA)

    def window_body(g, _):
        @pl.when(g % 2 == 0)
        def _():
            pltpu.make_async_copy(win_src(g), blk, semA).wait()

            @pl.when(g + 1 < nwin_j)
            def _():
                pltpu.async_copy(win_src(g + 1), blkB, semB)

            compute_win2(blk, g)

        @pl.when(g % 2 == 1)
        def _():
            pltpu.make_async_copy(win_src(g), blkB, semB).wait()

            @pl.when(g + 1 < nwin_j)
            def _():
                pltpu.async_copy(win_src(g + 1), blk, semA)

            compute_win2(blkB, g)

        return 0

    lax.fori_loop(0, nwin_j, window_body, 0)

    @pl.when(j == 7)
    def _():
        pltpu.sync_copy(
            logits_hbm.at[pl.ds(rb, 8), pl.ds(NWIN * WC, REMC)], blk2)
        for r in range(8):
            acc = blk2[r, pl.ds(0, L)]
            for v in range(1, REMC // L):
                acc = jnp.maximum(acc, blk2[r, pl.ds(v * L, L)])
            cm = jnp.max(acc)
            plsc.store_scatter(
                cmloc,
                [jnp.zeros((L,), jnp.int32) + (r * 256 + S7NW * (SW // CH))],
                jnp.zeros((L,), jnp.float32) + cm, mask=iota == 0)

    for r in range(8):
        def c_lt7(_, r=r):
            pltpu.sync_copy(
                cmloc.at[pl.ds(r * 256, SCH)],
                shared.at[pl.ds((lrb + r) * 2048 + j * SCH, SCH)])
            return 0

        def c_eq7(_, r=r):
            pltpu.sync_copy(
                cmloc.at[pl.ds(r * 256, S7W)],
                shared.at[pl.ds((lrb + r) * 2048 + 7 * SCH, S7W)])
            return 0

        lax.cond(j < 7, c_lt7, c_eq7, 0)

    plsc.subcore_barrier()

    pltpu.sync_copy(shared.at[pl.ds(sid * 2048, 2048)], cmax)

    def copy_body(i, _):
        cmx2[pl.ds(i * L, L)] = cmax[pl.ds(i * L, L)]
        return 0

    lax.fori_loop(0, NV_CM, copy_body, 0)

    def tau_iter(t, _):
        def sweep(i, a):
            return jnp.maximum(a, cmx2[pl.ds(i * L, L)])

        a = lax.fori_loop(0, NV_CM, sweep, negv)
        vstar = jnp.max(a)

        def mask_out(i, _):
            vv = cmx2[pl.ds(i * L, L)]
            cmx2[pl.ds(i * L, L)] = jnp.where(vv == vstar, negv, vv)
            return 0

        lax.fori_loop(0, NV_CM, mask_out, 0)
        return vstar

    tau = lax.fori_loop(0, K, tau_iter, jnp.float32(NEG))

    def init_cand(i, _):
        cval[pl.ds(i * L, L)] = negv
        cidx[pl.ds(i * L, L)] = bigv
        return 0

    lax.fori_loop(0, CAP // L, init_cand, 0)

    def append(ref, nv_, col0, cnt):
        def vreg(j, cnt):
            vv = ref[q, pl.ds(j * L, L)]
            m = vv >= tau
            mi = m.astype(jnp.int32)
            pos = cnt + plsc.cumsum(mi) - 1
            okm = m & (pos < CAP)
            gidx = col0 + j * L + iota
            plsc.store_scatter(cval, [pos], vv / TEMP, mask=okm)
            plsc.store_scatter(cidx, [pos], gidx, mask=okm)
            return cnt + jnp.sum(mi)

        return lax.fori_loop(0, nv_, vreg, cnt)

    def chunkb_vreg(i, cnt):
        cmv = cmax[pl.ds(i * L, L)]
        anyp = jnp.max(cmv)

        def scan_lanes(cnt):
            for lane in range(L):
                cml = cmv[lane]
                c = i * L + lane

                def do_full(cnt, c=c):
                    pltpu.sync_copy(
                        logits_hbm.at[pl.ds(rb, 8), pl.ds(c * CH, CH)], cbuf)
                    return append(cbuf, CH // L, c * CH, cnt)

                def do_rem(cnt):
                    pltpu.sync_copy(
                        logits_hbm.at[pl.ds(rb, 8), pl.ds(NWIN * WC, REMC)],
                        blk2)
                    return append(blk2, REMC // L, NWIN * WC, cnt)

                def fetch(cnt, c=c, do_full=do_full, do_rem=do_rem):
                    return lax.cond(c < NCHUNK - 1, do_full, do_rem, cnt)

                cnt = lax.cond(cml >= tau, fetch, lambda cnt: cnt, cnt)
            return cnt

        return lax.cond(anyp >= tau, scan_lanes, lambda cnt: cnt, cnt)

    cnt = lax.fori_loop(0, NV_CM, chunkb_vreg, jnp.int32(0))

    nv = (jnp.minimum(cnt, CAP) + (L - 1)) // L

    def ext(t, _):
        def sweep(i, a):
            return jnp.maximum(a, cval[pl.ds(i * L, L)])

        a = lax.fori_loop(0, nv, sweep, negv)
        vstar = jnp.max(a)

        def sweep2(i, a):
            vv = cval[pl.ds(i * L, L)]
            ix = cidx[pl.ds(i * L, L)]
            return jnp.minimum(a, jnp.where(vv == vstar, ix, bigv))

        iacc = lax.fori_loop(0, nv, sweep2, bigv)
        istar = jnp.min(iacc)

        def sweep3(i, _):
            vv = cval[pl.ds(i * L, L)]
            ix = cidx[pl.ds(i * L, L)]
            kill = (vv == vstar) & (ix == istar)
            cval[pl.ds(i * L, L)] = jnp.where(kill, negv, vv)
            return 0

        lax.fori_loop(0, nv, sweep3, 0)
        tv = jnp.zeros((L,), jnp.int32) + t
        plsc.store_scatter(sval, [tv],
                           jnp.zeros((L,), jnp.float32) + vstar,
                           mask=iota == 0)
        plsc.store_scatter(sidx, [tv],
                           jnp.zeros((L,), jnp.int32) + istar,
                           mask=iota == 0)
        return 0

    sval[pl.ds(48, L)] = negv
    sidx[pl.ds(48, L)] = bigv
    lax.fori_loop(0, K, ext, 0)

    m1 = sval[pl.ds(0, L)][0]
    evs = []
    s1acc = jnp.zeros((L,), jnp.float32)
    for b in range(4):
        e = jnp.exp(sval[pl.ds(b * L, L)] - m1)
        evs.append(e)
        s1acc = s1acc + e
    S1 = jnp.sum(s1acc)

    carry = jnp.float32(0.0)
    nkeep = jnp.int32(1)
    for b in range(4):
        cs = plsc.cumsum(evs[b] / S1) + carry
        carry = jnp.max(cs)
        tnum = b * L + iota
        nkeep = nkeep + jnp.sum(((cs <= TOPP) & (tnum < K - 1)).astype(jnp.int32))

    s2acc = jnp.zeros((L,), jnp.float32)
    eks = []
    for b in range(4):
        keep = (b * L + iota) < nkeep
        ek = jnp.where(keep, evs[b], jnp.float32(0.0))
        eks.append(ek)
        s2acc = s2acc + ek
    S2 = jnp.sum(s2acc)
    for b in range(4):
        pbuf[pl.ds(b * L, L)] = eks[b] / S2

    rrvv = rrv[pl.ds((wid // L) * L, L)]
    rr = jnp.max(jnp.where(iota == wid % L, rrvv, jnp.float32(NEG)))

    def ansb(t, ans):
        itv = sidx[pl.ds((t // L) * L, L)]
        it = jnp.min(jnp.where(iota == t % L, itv, bigv))
        acc = jnp.zeros((L,), jnp.float32)
        for b in range(4):
            pv = pbuf[pl.ds(b * L, L)]
            iv = sidx[pl.ds(b * L, L)]
            acc = acc + jnp.where(iv <= it, pv, jnp.float32(0.0))
        P = jnp.sum(acc)
        return jnp.where(P > rr, jnp.minimum(ans, it), ans)

    ans = lax.fori_loop(0, K, ansb, jnp.int32(V))

    outv[...] = jnp.zeros((L,), jnp.int32) + ans
    pltpu.sync_copy(outv, out_hbm.at[wid])


@jax.jit
def _sampler_sc(logits, rr_flat):
    f = functools.partial(
        pl.kernel,
        out_type=jax.ShapeDtypeStruct((ROWS, L), jnp.int32),
        mesh=plsc.VectorSubcoreMesh(core_axis_name="c", subcore_axis_name="s"),
        compiler_params=pltpu.CompilerParams(needs_layout_passes=False,
                                             use_tc_tiling_on_sc=True),
        scratch_types=[
            pltpu.VMEM((8, SW), jnp.float32),
            pltpu.VMEM((8, SW), jnp.float32),
            pltpu.VMEM((8, REMC), jnp.float32),
            pltpu.VMEM((2048,), jnp.float32),
            pltpu.VMEM((2048,), jnp.float32),
            pltpu.VMEM((2048,), jnp.float32),
            pltpu.VMEM_SHARED((32768,), jnp.float32),
            pltpu.VMEM((8, CH), jnp.float32),
            pltpu.VMEM((CAP,), jnp.float32),
            pltpu.VMEM((CAP,), jnp.int32),
            pltpu.VMEM((64,), jnp.float32),
            pltpu.VMEM((64,), jnp.int32),
            pltpu.VMEM((64,), jnp.float32),
            pltpu.VMEM((ROWS,), jnp.float32),
            pltpu.VMEM((L,), jnp.int32),
            pltpu.SemaphoreType.DMA,
            pltpu.SemaphoreType.DMA,
        ],
    )(_sampler_body)
    return f(logits, rr_flat)


def kernel(logits, rr):
    out16 = _sampler_sc(logits, rr.reshape(-1))
    return out16[:, :1]


# speedup vs baseline: 1.2817x; 1.0630x over previous
import functools

import jax
import jax.numpy as jnp
from jax import lax
from jax.experimental import pallas as pl
from jax.experimental.pallas import tpu as pltpu
from jax.experimental.pallas import tpu_sc as plsc

ROWS = 32
V = 1_000_000
WC = 4096
NWIN = V // WC
REMC = V - NWIN * WC
CH = 512
CPW = WC // CH
NCHUNK = NWIN * CPW + 1
NV_CM = (NCHUNK + 15) // 16
CMPAD = NV_CM * 16
CAP = 1024
K = 50
NEG = -3.0e38
BIG = 2**30
TEMP = 0.7
TOPP = 0.9
L = 16

SW = 4096
SCH = 248
STRIPE = SCH * CH
SNW = STRIPE // SW
S7NW = 27
S7W = 232


def _sampler_body(logits_hbm, rr_hbm, out_hbm,
                  blk, blkB, blk2, cmax, cmx2, cmloc, shared, cbuf, cval,
                  cidx, sval, sidx, pbuf, rrv, outv, semA, semB):
    sid = lax.axis_index("s")
    wid = lax.axis_index("c") * 16 + sid
    rb = (wid // 8) * 8
    q = wid % 8
    iota = lax.iota(jnp.int32, L)
    negv = jnp.full((L,), NEG, jnp.float32)
    bigv = jnp.full((L,), BIG, jnp.int32)

    pltpu.sync_copy(rr_hbm, rrv)

    j = sid % 8
    lrb = (sid // 8) * 8
    sbase = j * STRIPE
    nwin_j = jnp.where(j < 7, SNW, S7NW)

    def initcm(i, _):
        cmloc[pl.ds(i * L, L)] = negv
        return 0

    lax.fori_loop(0, 2048 // L, initcm, 0)

    def compute_win2(buf, g):
        def chunk_i(i, _):
            cb = i * CH
            gci = g * (SW // CH) + i
            for r in range(8):
                acc = buf[r, pl.ds(cb, L)]
                for v in range(1, CH // L):
                    acc = jnp.maximum(acc, buf[r, pl.ds(cb + v * L, L)])
                cm = jnp.max(acc)
                plsc.store_scatter(
                    cmloc,
                    [jnp.zeros((L,), jnp.int32) + (r * 256 + gci)],
                    jnp.zeros((L,), jnp.float32) + cm, mask=iota == 0)
            return 0

        lax.fori_loop(0, SW // CH, chunk_i, 0)

    def win_src(g):
        return logits_hbm.at[pl.ds(rb, 8), pl.ds(sbase + g * SW, SW)]

    pltpu.async_copy(win_src(0), blk, semA)

    def window_body(g, _):
        @pl.when(g % 2 == 0)
        def _():
            pltpu.make_async_copy(win_src(g), blk, semA).wait()

            @pl.when(g + 1 < nwin_j)
            def _():
                pltpu.async_copy(win_src(g + 1), blkB, semB)

            compute_win2(blk, g)

        @pl.when(g % 2 == 1)
        def _():
            pltpu.make_async_copy(win_src(g), blkB, semB).wait()

            @pl.when(g + 1 < nwin_j)
            def _():
                pltpu.async_copy(win_src(g + 1), blk, semA)

            compute_win2(blkB, g)

        return 0

    lax.fori_loop(0, nwin_j, window_body, 0)

    @pl.when(j == 7)
    def _():
        pltpu.sync_copy(
            logits_hbm.at[pl.ds(rb, 8), pl.ds(NWIN * WC, REMC)], blk2)
        for r in range(8):
            acc = blk2[r, pl.ds(0, L)]
            for v in range(1, REMC // L):
                acc = jnp.maximum(acc, blk2[r, pl.ds(v * L, L)])
            cm = jnp.max(acc)
            plsc.store_scatter(
                cmloc,
                [jnp.zeros((L,), jnp.int32) + (r * 256 + S7NW * (SW // CH))],
                jnp.zeros((L,), jnp.float32) + cm, mask=iota == 0)

    for r in range(8):
        def c_lt7(_, r=r):
            pltpu.sync_copy(
                cmloc.at[pl.ds(r * 256, SCH)],
                shared.at[pl.ds((lrb + r) * 2048 + j * SCH, SCH)])
            return 0

        def c_eq7(_, r=r):
            pltpu.sync_copy(
                cmloc.at[pl.ds(r * 256, S7W)],
                shared.at[pl.ds((lrb + r) * 2048 + 7 * SCH, S7W)])
            return 0

        lax.cond(j < 7, c_lt7, c_eq7, 0)

    plsc.subcore_barrier()

    pltpu.sync_copy(shared.at[pl.ds(sid * 2048, 2048)], cmax)

    def copy_body(i, _):
        cmx2[pl.ds(i * L, L)] = cmax[pl.ds(i * L, L)]
        return 0

    lax.fori_loop(0, NV_CM, copy_body, 0)

    def tau_iter(t, _):
        def sweep(i, a):
            return jnp.maximum(a, cmx2[pl.ds(i * L, L)])

        a = lax.fori_loop(0, NV_CM, sweep, negv)
        vstar = jnp.max(a)

        def mask_out(i, _):
            vv = cmx2[pl.ds(i * L, L)]
            cmx2[pl.ds(i * L, L)] = jnp.where(vv == vstar, negv, vv)
            return 0

        lax.fori_loop(0, NV_CM, mask_out, 0)
        return vstar

    tau = lax.fori_loop(0, K, tau_iter, jnp.float32(NEG))

    def init_cand(i, _):
        cval[pl.ds(i * L, L)] = negv
        cidx[pl.ds(i * L, L)] = bigv
        return 0

    lax.fori_loop(0, CAP // L, init_cand, 0)

    def append(ref, nv_, col0, cnt):
        def vreg(j, cnt):
            vv = ref[q, pl.ds(j * L, L)]
            m = vv >= tau
            mi = m.astype(jnp.int32)
            pos = cnt + plsc.cumsum(mi) - 1
            okm = m & (pos < CAP)
            gidx = col0 + j * L + iota
            plsc.store_scatter(cval, [pos], vv / TEMP, mask=okm)
            plsc.store_scatter(cidx, [pos], gidx, mask=okm)
            return cnt + jnp.sum(mi)

        return lax.fori_loop(0, nv_, vreg, cnt)

    def chunkb_vreg(i, cnt):
        cmv = cmax[pl.ds(i * L, L)]
        anyp = jnp.max(cmv)

        def scan_lanes(cnt):
            for lane in range(L):
                cml = cmv[lane]
                c = i * L + lane

                def do_full(cnt, c=c):
                    pltpu.sync_copy(
                        logits_hbm.at[pl.ds(rb, 8), pl.ds(c * CH, CH)], cbuf)
                    return append(cbuf, CH // L, c * CH, cnt)

                def do_rem(cnt):
                    pltpu.sync_copy(
                        logits_hbm.at[pl.ds(rb, 8), pl.ds(NWIN * WC, REMC)],
                        blk2)
                    return append(blk2, REMC // L, NWIN * WC, cnt)

                def fetch(cnt, c=c, do_full=do_full, do_rem=do_rem):
                    return lax.cond(c < NCHUNK - 1, do_full, do_rem, cnt)

                cnt = lax.cond(cml >= tau, fetch, lambda cnt: cnt, cnt)
            return cnt

        return lax.cond(anyp >= tau, scan_lanes, lambda cnt: cnt, cnt)

    cnt = lax.fori_loop(0, NV_CM, chunkb_vreg, jnp.int32(0))

    nv = (jnp.minimum(cnt, CAP) + (L - 1)) // L

    def ext(t, _):
        def sweep(i, a):
            return jnp.maximum(a, cval[pl.ds(i * L, L)])

        a = lax.fori_loop(0, nv, sweep, negv)
        vstar = jnp.max(a)

        def sweep2(i, a):
            vv = cval[pl.ds(i * L, L)]
            ix = cidx[pl.ds(i * L, L)]
            return jnp.minimum(a, jnp.where(vv == vstar, ix, bigv))

        iacc = lax.fori_loop(0, nv, sweep2, bigv)
        istar = jnp.min(iacc)

        def sweep3(i, _):
            vv = cval[pl.ds(i * L, L)]
            ix = cidx[pl.ds(i * L, L)]
            kill = (vv == vstar) & (ix == istar)
            cval[pl.ds(i * L, L)] = jnp.where(kill, negv, vv)
            return 0

        lax.fori_loop(0, nv, sweep3, 0)
        tv = jnp.zeros((L,), jnp.int32) + t
        plsc.store_scatter(sval, [tv],
                           jnp.zeros((L,), jnp.float32) + vstar,
                           mask=iota == 0)
        plsc.store_scatter(sidx, [tv],
                           jnp.zeros((L,), jnp.int32) + istar,
                           mask=iota == 0)
        return 0

    sval[pl.ds(48, L)] = negv
    sidx[pl.ds(48, L)] = bigv
    lax.fori_loop(0, K, ext, 0)

    m1 = sval[pl.ds(0, L)][0]
    evs = []
    s1acc = jnp.zeros((L,), jnp.float32)
    for b in range(4):
        e = jnp.exp(sval[pl.ds(b * L, L)] - m1)
        evs.append(e)
        s1acc = s1acc + e
    S1 = jnp.sum(s1acc)

    carry = jnp.float32(0.0)
    nkeep = jnp.int32(1)
    for b in range(4):
        cs = plsc.cumsum(evs[b] / S1) + carry
        carry = jnp.max(cs)
        tnum = b * L + iota
        nkeep = nkeep + jnp.sum(((cs <= TOPP) & (tnum < K - 1)).astype(jnp.int32))

    s2acc = jnp.zeros((L,), jnp.float32)
    eks = []
    for b in range(4):
        keep = (b * L + iota) < nkeep
        ek = jnp.where(keep, evs[b], jnp.float32(0.0))
        eks.append(ek)
        s2acc = s2acc + ek
    S2 = jnp.sum(s2acc)
    for b in range(4):
        pbuf[pl.ds(b * L, L)] = eks[b] / S2

    rrvv = rrv[pl.ds((wid // L) * L, L)]
    rr = jnp.max(jnp.where(iota == wid % L, rrvv, jnp.float32(NEG)))

    def ansb(t, ans):
        itv = sidx[pl.ds((t // L) * L, L)]
        it = jnp.min(jnp.where(iota == t % L, itv, bigv))
        acc = jnp.zeros((L,), jnp.float32)
        for b in range(4):
            pv = pbuf[pl.ds(b * L, L)]
            iv = sidx[pl.ds(b * L, L)]
            acc = acc + jnp.where(iv <= it, pv, jnp.float32(0.0))
        P = jnp.sum(acc)
        return jnp.where(P > rr, jnp.minimum(ans, it), ans)

    ans = lax.fori_loop(0, K, ansb, jnp.int32(V))

    outv[...] = jnp.zeros((L,), jnp.int32) + ans
    pltpu.sync_copy(outv, out_hbm.at[wid])


@jax.jit
def _sampler_sc(logits, rr_flat):
    f = functools.partial(
        pl.kernel,
        out_type=jax.ShapeDtypeStruct((ROWS, L), jnp.int32),
        mesh=plsc.VectorSubcoreMesh(core_axis_name="c", subcore_axis_name="s"),
        compiler_params=pltpu.CompilerParams(needs_layout_passes=False,
                                             use_tc_tiling_on_sc=True),
        scratch_types=[
            pltpu.VMEM((8, SW), jnp.float32),
            pltpu.VMEM((8, SW), jnp.float32),
            pltpu.VMEM((8, REMC), jnp.float32),
            pltpu.VMEM((2048,), jnp.float32),
            pltpu.VMEM((2048,), jnp.float32),
            pltpu.VMEM((2048,), jnp.float32),
            pltpu.VMEM_SHARED((32768,), jnp.float32),
            pltpu.VMEM((8, CH), jnp.float32),
            pltpu.VMEM((CAP,), jnp.float32),
            pltpu.VMEM((CAP,), jnp.int32),
            pltpu.VMEM((64,), jnp.float32),
            pltpu.VMEM((64,), jnp.int32),
            pltpu.VMEM((64,), jnp.float32),
            pltpu.VMEM((ROWS,), jnp.float32),
            pltpu.VMEM((L,), jnp.int32),
            pltpu.SemaphoreType.DMA,
            pltpu.SemaphoreType.DMA,
        ],
    )(_sampler_body)
    return f(logits, rr_flat)


def kernel(logits, rr):
    out16 = _sampler_sc(logits, rr.reshape(-1))
    return out16[:, :1]


# speedup vs baseline: 1.3825x; 1.0787x over previous
import functools

import jax
import jax.numpy as jnp
from jax import lax
from jax.experimental import pallas as pl
from jax.experimental.pallas import tpu as pltpu
from jax.experimental.pallas import tpu_sc as plsc

ROWS = 32
V = 1_000_000
WC = 4096
NWIN = V // WC
REMC = V - NWIN * WC
CH = 512
CPW = WC // CH
NCHUNK = NWIN * CPW + 1
NV_CM = (NCHUNK + 15) // 16
CMPAD = NV_CM * 16
CAP = 1024
K = 50
NEG = -3.0e38
BIG = 2**30
TEMP = 0.7
TOPP = 0.9
L = 16

SW = 4096
SCH = 248
STRIPE = SCH * CH
SNW = STRIPE // SW
S7NW = 27
S7W = 232


def _sampler_body(logits_hbm, rr_hbm, out_hbm,
                  blk, blkB, blk2, cmax, cmx2, cmloc, shared, cbuf, cval,
                  cidx, sval, sidx, pbuf, rrv, outv, semA, semB):
    sid = lax.axis_index("s")
    wid = lax.axis_index("c") * 16 + sid
    rb = (wid // 8) * 8
    q = wid % 8
    iota = lax.iota(jnp.int32, L)
    negv = jnp.full((L,), NEG, jnp.float32)
    bigv = jnp.full((L,), BIG, jnp.int32)

    pltpu.sync_copy(rr_hbm, rrv)

    j = sid % 8
    lrb = (sid // 8) * 8
    sbase = j * STRIPE
    nwin_j = jnp.where(j < 7, SNW, S7NW)

    def initcm(i, _):
        cmloc[pl.ds(i * L, L)] = negv
        return 0

    lax.fori_loop(0, 2048 // L, initcm, 0)

    def compute_win2(buf, g):
        def chunk_i(i, _):
            cb = i * CH
            gci = g * (SW // CH) + i
            for r in range(8):
                acc = buf[r, pl.ds(cb, L)]
                for v in range(1, CH // L):
                    acc = jnp.maximum(acc, buf[r, pl.ds(cb + v * L, L)])
                cm = jnp.max(acc)
                plsc.store_scatter(
                    cmloc,
                    [jnp.zeros((L,), jnp.int32) + (r * 256 + gci)],
                    jnp.zeros((L,), jnp.float32) + cm, mask=iota == 0)
            return 0

        lax.fori_loop(0, SW // CH, chunk_i, 0)

    def win_src(g):
        return logits_hbm.at[pl.ds(rb, 8), pl.ds(sbase + g * SW, SW)]

    pltpu.async_copy(win_src(0), blk, semA)

    def window_body(g, _):
        @pl.when(g % 2 == 0)
        def _():
            pltpu.make_async_copy(win_src(g), blk, semA).wait()

            @pl.when(g + 1 < nwin_j)
            def _():
                pltpu.async_copy(win_src(g + 1), blkB, semB)

            compute_win2(blk, g)

        @pl.when(g % 2 == 1)
        def _():
            pltpu.make_async_copy(win_src(g), blkB, semB).wait()

            @pl.when(g + 1 < nwin_j)
            def _():
                pltpu.async_copy(win_src(g + 1), blk, semA)

            compute_win2(blkB, g)

        return 0

    lax.fori_loop(0, nwin_j, window_body, 0)

    @pl.when(j == 7)
    def _():
        pltpu.sync_copy(
            logits_hbm.at[pl.ds(rb, 8), pl.ds(NWIN * WC, REMC)], blk2)
        for r in range(8):
            acc = blk2[r, pl.ds(0, L)]
            for v in range(1, REMC // L):
                acc = jnp.maximum(acc, blk2[r, pl.ds(v * L, L)])
            cm = jnp.max(acc)
            plsc.store_scatter(
                cmloc,
                [jnp.zeros((L,), jnp.int32) + (r * 256 + S7NW * (SW // CH))],
                jnp.zeros((L,), jnp.float32) + cm, mask=iota == 0)

    for r in range(8):
        def c_lt7(_, r=r):
            pltpu.sync_copy(
                cmloc.at[pl.ds(r * 256, SCH)],
                shared.at[pl.ds((lrb + r) * 2048 + j * SCH, SCH)])
            return 0

        def c_eq7(_, r=r):
            pltpu.sync_copy(
                cmloc.at[pl.ds(r * 256, S7W)],
                shared.at[pl.ds((lrb + r) * 2048 + 7 * SCH, S7W)])
            return 0

        lax.cond(j < 7, c_lt7, c_eq7, 0)

    plsc.subcore_barrier()

    pltpu.sync_copy(shared.at[pl.ds(sid * 2048, 2048)], cmax)

    def copy_body(i, _):
        cmx2[pl.ds(i * L, L)] = cmax[pl.ds(i * L, L)]
        return 0

    lax.fori_loop(0, NV_CM, copy_body, 0)

    def sweep0(i, a):
        return jnp.maximum(a, cmx2[pl.ds(i * L, L)])

    vstar0 = jnp.max(lax.fori_loop(0, NV_CM, sweep0, negv))

    def tau_iter(t, vstar):
        def body(i, a):
            vv = cmx2[pl.ds(i * L, L)]
            vv2 = jnp.where(vv == vstar, negv, vv)
            cmx2[pl.ds(i * L, L)] = vv2
            return jnp.maximum(a, vv2)

        a = lax.fori_loop(0, NV_CM, body, negv)
        return jnp.max(a)

    tau = lax.fori_loop(0, K - 1, tau_iter, vstar0)

    def init_cand(i, _):
        cval[pl.ds(i * L, L)] = negv
        cidx[pl.ds(i * L, L)] = bigv
        return 0

    lax.fori_loop(0, CAP // L, init_cand, 0)

    def append(ref, nv_, col0, cnt):
        def vreg(j, cnt):
            vv = ref[q, pl.ds(j * L, L)]
            m = vv >= tau
            mi = m.astype(jnp.int32)
            pos = cnt + plsc.cumsum(mi) - 1
            okm = m & (pos < CAP)
            gidx = col0 + j * L + iota
            plsc.store_scatter(cval, [pos], vv / TEMP, mask=okm)
            plsc.store_scatter(cidx, [pos], gidx, mask=okm)
            return cnt + jnp.sum(mi)

        return lax.fori_loop(0, nv_, vreg, cnt)

    def chunkb_vreg(i, cnt):
        cmv = cmax[pl.ds(i * L, L)]
        anyp = jnp.max(cmv)

        def scan_lanes(cnt):
            for lane in range(L):
                cml = cmv[lane]
                c = i * L + lane

                def do_full(cnt, c=c):
                    pltpu.sync_copy(
                        logits_hbm.at[pl.ds(rb, 8), pl.ds(c * CH, CH)], cbuf)
                    return append(cbuf, CH // L, c * CH, cnt)

                def do_rem(cnt):
                    pltpu.sync_copy(
                        logits_hbm.at[pl.ds(rb, 8), pl.ds(NWIN * WC, REMC)],
                        blk2)
                    return append(blk2, REMC // L, NWIN * WC, cnt)

                def fetch(cnt, c=c, do_full=do_full, do_rem=do_rem):
                    return lax.cond(c < NCHUNK - 1, do_full, do_rem, cnt)

                cnt = lax.cond(cml >= tau, fetch, lambda cnt: cnt, cnt)
            return cnt

        return lax.cond(anyp >= tau, scan_lanes, lambda cnt: cnt, cnt)

    cnt = lax.fori_loop(0, NV_CM, chunkb_vreg, jnp.int32(0))

    nv = (jnp.minimum(cnt, CAP) + (L - 1)) // L

    def ext(t, _):
        def sweep(i, a):
            return jnp.maximum(a, cval[pl.ds(i * L, L)])

        a = lax.fori_loop(0, nv, sweep, negv)
        vstar = jnp.max(a)

        def sweep2(i, a):
            vv = cval[pl.ds(i * L, L)]
            ix = cidx[pl.ds(i * L, L)]
            return jnp.minimum(a, jnp.where(vv == vstar, ix, bigv))

        iacc = lax.fori_loop(0, nv, sweep2, bigv)
        istar = jnp.min(iacc)

        def sweep3(i, _):
            vv = cval[pl.ds(i * L, L)]
            ix = cidx[pl.ds(i * L, L)]
            kill = (vv == vstar) & (ix == istar)
            cval[pl.ds(i * L, L)] = jnp.where(kill, negv, vv)
            return 0

        lax.fori_loop(0, nv, sweep3, 0)
        tv = jnp.zeros((L,), jnp.int32) + t
        plsc.store_scatter(sval, [tv],
                           jnp.zeros((L,), jnp.float32) + vstar,
                           mask=iota == 0)
        plsc.store_scatter(sidx, [tv],
                           jnp.zeros((L,), jnp.int32) + istar,
                           mask=iota == 0)
        return 0

    sval[pl.ds(48, L)] = negv
    sidx[pl.ds(48, L)] = bigv
    lax.fori_loop(0, K, ext, 0)

    m1 = sval[pl.ds(0, L)][0]
    evs = []
    s1acc = jnp.zeros((L,), jnp.float32)
    for b in range(4):
        e = jnp.exp(sval[pl.ds(b * L, L)] - m1)
        evs.append(e)
        s1acc = s1acc + e
    S1 = jnp.sum(s1acc)

    carry = jnp.float32(0.0)
    nkeep = jnp.int32(1)
    for b in range(4):
        cs = plsc.cumsum(evs[b] / S1) + carry
        carry = jnp.max(cs)
        tnum = b * L + iota
        nkeep = nkeep + jnp.sum(((cs <= TOPP) & (tnum < K - 1)).astype(jnp.int32))

    s2acc = jnp.zeros((L,), jnp.float32)
    eks = []
    for b in range(4):
        keep = (b * L + iota) < nkeep
        ek = jnp.where(keep, evs[b], jnp.float32(0.0))
        eks.append(ek)
        s2acc = s2acc + ek
    S2 = jnp.sum(s2acc)
    for b in range(4):
        pbuf[pl.ds(b * L, L)] = eks[b] / S2

    rrvv = rrv[pl.ds((wid // L) * L, L)]
    rr = jnp.max(jnp.where(iota == wid % L, rrvv, jnp.float32(NEG)))

    def ansb(t, ans):
        itv = sidx[pl.ds((t // L) * L, L)]
        it = jnp.min(jnp.where(iota == t % L, itv, bigv))
        acc = jnp.zeros((L,), jnp.float32)
        for b in range(4):
            pv = pbuf[pl.ds(b * L, L)]
            iv = sidx[pl.ds(b * L, L)]
            acc = acc + jnp.where(iv <= it, pv, jnp.float32(0.0))
        P = jnp.sum(acc)
        return jnp.where(P > rr, jnp.minimum(ans, it), ans)

    ans = lax.fori_loop(0, K, ansb, jnp.int32(V))

    outv[...] = jnp.zeros((L,), jnp.int32) + ans
    pltpu.sync_copy(outv, out_hbm.at[wid])


@jax.jit
def _sampler_sc(logits, rr_flat):
    f = functools.partial(
        pl.kernel,
        out_type=jax.ShapeDtypeStruct((ROWS, L), jnp.int32),
        mesh=plsc.VectorSubcoreMesh(core_axis_name="c", subcore_axis_name="s"),
        compiler_params=pltpu.CompilerParams(needs_layout_passes=False,
                                             use_tc_tiling_on_sc=True),
        scratch_types=[
            pltpu.VMEM((8, SW), jnp.float32),
            pltpu.VMEM((8, SW), jnp.float32),
            pltpu.VMEM((8, REMC), jnp.float32),
            pltpu.VMEM((2048,), jnp.float32),
            pltpu.VMEM((2048,), jnp.float32),
            pltpu.VMEM((2048,), jnp.float32),
            pltpu.VMEM_SHARED((32768,), jnp.float32),
            pltpu.VMEM((8, CH), jnp.float32),
            pltpu.VMEM((CAP,), jnp.float32),
            pltpu.VMEM((CAP,), jnp.int32),
            pltpu.VMEM((64,), jnp.float32),
            pltpu.VMEM((64,), jnp.int32),
            pltpu.VMEM((64,), jnp.float32),
            pltpu.VMEM((ROWS,), jnp.float32),
            pltpu.VMEM((L,), jnp.int32),
            pltpu.SemaphoreType.DMA,
            pltpu.SemaphoreType.DMA,
        ],
    )(_sampler_body)
    return f(logits, rr_flat)


def kernel(logits, rr):
    out16 = _sampler_sc(logits, rr.reshape(-1))
    return out16[:, :1]
